# baseline (speedup 1.0000x reference)
import functools

import jax
import jax.numpy as jnp
from jax import lax
from jax.experimental import pallas as pl
from jax.experimental.pallas import tpu as pltpu
from jax.experimental.pallas import tpu_sc as plsc

_B = 16384
_NC, _NS = 2, 16
_NW = _NC * _NS
_BPW = _B // _NW
_CH = 128
_NCH = _BPW // _CH

_DOM_D, _AUTH_D, _USER_D, _ITEM_D = 16, 16, 16, 8
_HIDDEN = 128


def _sc_gather(dom_i, auth_i, user_i, item_i, dom_t, auth_t, user_t, item_t):
    mesh = plsc.VectorSubcoreMesh(core_axis_name="c", subcore_axis_name="s")

    @functools.partial(
        pl.kernel,
        mesh=mesh,
        out_type=[
            jax.ShapeDtypeStruct((_B, _DOM_D), jnp.float32),
            jax.ShapeDtypeStruct((_B, _AUTH_D), jnp.float32),
            jax.ShapeDtypeStruct((_B, _USER_D), jnp.float32),
            jax.ShapeDtypeStruct((_B, _ITEM_D), jnp.float32),
        ],
        scratch_types=[
            pltpu.VMEM((_NCH, _CH), jnp.int32),
            pltpu.VMEM((_NCH, _CH), jnp.int32),
            pltpu.VMEM((_NCH, _CH), jnp.int32),
            pltpu.VMEM((_NCH, _CH), jnp.int32),
            pltpu.VMEM((_BPW, _DOM_D), jnp.float32),
            pltpu.VMEM((_BPW, _AUTH_D), jnp.float32),
            pltpu.VMEM((_BPW, _USER_D), jnp.float32),
            pltpu.VMEM((_BPW, _ITEM_D), jnp.float32),
            pltpu.SemaphoreType.DMA,
        ],
        compiler_params=pltpu.CompilerParams(use_tc_tiling_on_sc=False),
    )
    def k(di, ai, ui, ii, dt, at_, ut, it,
          o_d, o_a, o_u, o_i, xd, xa, xu, xi, rd, ra, ru, ri, sem):
        wid = lax.axis_index("s") * _NC + lax.axis_index("c")
        base = wid * _BPW
        row = wid * _NCH
        idx_loads = [
            pltpu.async_copy(di.at[pl.ds(row, _NCH)], xd, sem),
            pltpu.async_copy(ai.at[pl.ds(row, _NCH)], xa, sem),
            pltpu.async_copy(ui.at[pl.ds(row, _NCH)], xu, sem),
            pltpu.async_copy(ii.at[pl.ds(row, _NCH)], xi, sem),
        ]
        for c in idx_loads:
            c.wait()
        gathers = []
        for j in range(_NCH):
            sl = pl.ds(j * _CH, _CH)
            gathers.append(pltpu.async_copy(dt.at[xd.at[j]], rd.at[sl], sem))
            gathers.append(pltpu.async_copy(at_.at[xa.at[j]], ra.at[sl], sem))
            gathers.append(pltpu.async_copy(ut.at[xu.at[j]], ru.at[sl], sem))
            gathers.append(pltpu.async_copy(it.at[xi.at[j]], ri.at[sl], sem))
        for c in gathers:
            c.wait()
        out_sl = pl.ds(base, _BPW)
        pltpu.sync_copy(rd, o_d.at[out_sl])
        pltpu.sync_copy(ra, o_a.at[out_sl])
        pltpu.sync_copy(ru, o_u.at[out_sl])
        pltpu.sync_copy(ri, o_i.at[out_sl])

    return k(dom_i, auth_i, user_i, item_i, dom_t, auth_t, user_t, item_t)


_BLK = 2048


def _mlp_body(t_ref, dy_ref, nm_ref, d_ref, a_ref, u_ref, i_ref,
              tt_ref, dt_ref, w1_ref, b1_ref, w2_ref, b2_ref, o_ref):
    t_oh = (lax.broadcasted_iota(jnp.int32, (_BLK, 8), 1) == t_ref[...]
            ).astype(jnp.float32)
    dy_oh = (lax.broadcasted_iota(jnp.int32, (_BLK, 8), 1) == dy_ref[...]
             ).astype(jnp.float32)
    type_emb = jnp.dot(t_oh, tt_ref[...], preferred_element_type=jnp.float32)
    day_emb = jnp.dot(dy_oh, dt_ref[...], preferred_element_type=jnp.float32)
    x = jnp.concatenate(
        [type_emb, day_emb, d_ref[...], a_ref[...], u_ref[...], i_ref[...],
         nm_ref[...]], axis=1)
    h = jnp.maximum(
        jnp.dot(x, w1_ref[...], preferred_element_type=jnp.float32)
        + b1_ref[...], 0.0)
    o_ref[...] = (jnp.dot(h, w2_ref[...], preferred_element_type=jnp.float32)
                  + b2_ref[...])


def _tc_mlp(t2, d2, num3, dom, auth, user, item, type_t, day_t8,
            W1, b1, W2, b2):
    grid = (_B // _BLK,)
    full = lambda shape: pl.BlockSpec(shape, lambda i: (0, 0))
    blk = lambda w: pl.BlockSpec((_BLK, w), lambda i: (i, 0))
    return pl.pallas_call(
        _mlp_body,
        grid=grid,
        in_specs=[
            blk(1), blk(1), blk(3),
            blk(_DOM_D), blk(_AUTH_D), blk(_USER_D), blk(_ITEM_D),
            full((8, 8)), full((8, 4)),
            full((71, _HIDDEN)), full((1, _HIDDEN)),
            full((_HIDDEN, 1)), full((1, 1)),
        ],
        out_specs=blk(1),
        out_shape=jax.ShapeDtypeStruct((_B, 1), jnp.float32),
    )(t2, d2, num3, dom, auth, user, item, type_t, day_t8, W1, b1, W2, b2)


def kernel(type_id, day_of_week_id, domain_id, author_id, user_id, item_id,
           hour_of_day, karma, descendants,
           type_table, day_table, domain_table, author_table, user_table,
           item_table, W1, b1, W2, b2):
    dom_i = domain_id.astype(jnp.int32).reshape(_B // _CH, _CH)
    auth_i = author_id.astype(jnp.int32).reshape(_B // _CH, _CH)
    user_i = user_id.astype(jnp.int32).reshape(_B // _CH, _CH)
    item_i = item_id.astype(jnp.int32).reshape(_B // _CH, _CH)
    dom, auth, user, item = _sc_gather(
        dom_i, auth_i, user_i, item_i,
        domain_table, author_table, user_table, item_table)

    num3 = jnp.stack([hour_of_day.astype(jnp.float32),
                      karma.astype(jnp.float32),
                      descendants.astype(jnp.float32)], axis=1)
    t2 = type_id.astype(jnp.int32).reshape(_B, 1)
    d2 = day_of_week_id.astype(jnp.int32).reshape(_B, 1)
    day_t8 = jnp.zeros((8, 4), day_table.dtype).at[:7].set(day_table)
    out = _tc_mlp(t2, d2, num3, dom, auth, user, item,
                  type_table, day_t8,
                  W1, b1.reshape(1, _HIDDEN), W2, b2.reshape(1, 1))
    return out.reshape(_B)
